# baseline (speedup 1.0000x reference)
import jax
import jax.numpy as jnp
from jax.experimental import pallas as pl
from jax.experimental.pallas import tpu as pltpu

_HIDDEN = 768
_EXPERTS = 64
_CHUNK = 1024
_NBUF = 4


def _router_kernel(x_hbm, w_ref, o_ref, xbuf, sems):
    m = x_hbm.shape[0]
    nchunks = m // _CHUNK

    def issue(slot, chunk):
        pltpu.make_async_copy(
            x_hbm.at[pl.ds(chunk * _CHUNK, _CHUNK), :],
            xbuf.at[slot],
            sems.at[slot],
        ).start()

    for b in range(_NBUF):
        issue(b, b)

    def body(i, carry):
        slot = jax.lax.rem(i, _NBUF)
        pltpu.make_async_copy(
            x_hbm.at[pl.ds(i * _CHUNK, _CHUNK), :],
            xbuf.at[slot],
            sems.at[slot],
        ).wait()
        acc = jax.lax.dot_general(
            xbuf[slot],
            w_ref[...],
            dimension_numbers=(((1,), (1,)), ((), ())),
            preferred_element_type=jnp.float32,
        )
        o_ref[pl.ds(i * _CHUNK, _CHUNK), :] = acc

        nxt = i + _NBUF

        @pl.when(nxt < nchunks)
        def _():
            issue(slot, nxt)

        return carry

    jax.lax.fori_loop(0, nchunks, body, 0)


@jax.jit
def kernel(hidden_states, W):
    x = hidden_states.reshape(-1, _HIDDEN)
    m = x.shape[0]
    return pl.pallas_call(
        _router_kernel,
        in_specs=[
            pl.BlockSpec(memory_space=pl.ANY),
            pl.BlockSpec(memory_space=pltpu.VMEM),
        ],
        out_specs=pl.BlockSpec(memory_space=pltpu.VMEM),
        out_shape=jax.ShapeDtypeStruct((m, _EXPERTS), jnp.float32),
        scratch_shapes=[
            pltpu.VMEM((_NBUF, _CHUNK, _HIDDEN), jnp.float32),
            pltpu.SemaphoreType.DMA((_NBUF,)),
        ],
    )(x, W)
